# baseline (speedup 1.0000x reference)
import functools

import jax
import jax.numpy as jnp
from jax import lax
from jax.experimental import pallas as pl
from jax.experimental.pallas import tpu as pltpu
from jax.experimental.pallas import tpu_sc as plsc

B, S, H, W = 4, 2, 512, 512
K = 8
N = H * W
BN = B * N
DELTA_PUSH = 20.0
W_PULL, W_PUSH, W_PEN, W_BEN = 1.0, 1.0, 1.0, 5.0

BH = 64
R = H // BH


def _dense_body(off_x, off_y, nr_x, nr_y, gr_x, gr_y, lab,
                misc_out, sx_out, sy_out, cnt_out,
                acc_misc, acc_sx, acc_sy, acc_cnt):
    b = pl.program_id(0)
    r = pl.program_id(1)

    @pl.when(jnp.logical_and(b == 0, r == 0))
    def _init():
        acc_misc[...] = jnp.zeros_like(acc_misc)

    @pl.when(r == 0)
    def _init_batch():
        acc_sx[...] = jnp.zeros_like(acc_sx)
        acc_sy[...] = jnp.zeros_like(acc_sy)
        acc_cnt[...] = jnp.zeros_like(acc_cnt)

    ox = off_x[0, 0]
    oy = off_y[0, 0]
    nx = nr_x[0, 0]
    ny = nr_y[0, 0]
    gx_ = gr_x[0, 0]
    gy_ = gr_y[0, 0]
    labs = lab[0]

    col = lax.broadcasted_iota(jnp.int32, (BH, W), 1).astype(jnp.float32)
    row = (lax.broadcasted_iota(jnp.int32, (BH, W), 0) + r * BH).astype(jnp.float32)
    ex = col + ox
    ey = row + oy

    fg = (labs > 0).astype(jnp.float32)

    d2 = (ex - nx) ** 2 + (ey - ny) ** 2 + 1e-8
    dist = jnp.sqrt(d2)
    acc_misc[0:1, :] += jnp.sum(dist * fg, axis=0, keepdims=True)

    off_n = jnp.sqrt(ox * ox + oy * oy + 1e-12)
    grad_n = jnp.sqrt(gx_ * gx_ + gy_ * gy_ + 1e-12)
    cos = (ox * gx_ + oy * gy_) / (off_n * grad_n + 1e-8)
    acc_misc[1:2, :] += jnp.sum((1.0 - cos) * fg, axis=0, keepdims=True)

    acc_misc[2:3, :] += jnp.sum(fg, axis=0, keepdims=True)

    for k in range(K):
        m = (labs == k).astype(jnp.float32)
        acc_sx[k:k + 1, :] += jnp.sum(m * ex, axis=0, keepdims=True)
        acc_sy[k:k + 1, :] += jnp.sum(m * ey, axis=0, keepdims=True)
        acc_cnt[k:k + 1, :] += jnp.sum(m, axis=0, keepdims=True)

    @pl.when(r == R - 1)
    def _flush_batch():
        sx_out[0] = acc_sx[...]
        sy_out[0] = acc_sy[...]
        cnt_out[0] = acc_cnt[...]

    @pl.when(jnp.logical_and(b == B - 1, r == R - 1))
    def _flush():
        misc_out[...] = acc_misc[...]


def _dense_terms(offsets, gt_labels, gt_nr_skel, gt_dt_grad):
    grid = (B, R)
    blk4 = (1, 1, BH, W)

    def chan(c):
        return pl.BlockSpec(blk4, lambda b, r, c=c: (b, c, r, 0))

    lab_spec = pl.BlockSpec((1, BH, W), lambda b, r: (b, r, 0))

    out_shapes = [
        jax.ShapeDtypeStruct((8, W), jnp.float32),
        jax.ShapeDtypeStruct((B, K, W), jnp.float32),
        jax.ShapeDtypeStruct((B, K, W), jnp.float32),
        jax.ShapeDtypeStruct((B, K, W), jnp.float32),
    ]
    out_specs = [
        pl.BlockSpec((8, W), lambda b, r: (0, 0)),
        pl.BlockSpec((1, K, W), lambda b, r: (b, 0, 0)),
        pl.BlockSpec((1, K, W), lambda b, r: (b, 0, 0)),
        pl.BlockSpec((1, K, W), lambda b, r: (b, 0, 0)),
    ]
    return pl.pallas_call(
        _dense_body,
        grid=grid,
        in_specs=[chan(0), chan(1), chan(0), chan(1), chan(0), chan(1), lab_spec],
        out_specs=out_specs,
        out_shape=out_shapes,
        scratch_shapes=[
            pltpu.VMEM((8, W), jnp.float32),
            pltpu.VMEM((K, W), jnp.float32),
            pltpu.VMEM((K, W), jnp.float32),
            pltpu.VMEM((K, W), jnp.float32),
        ],
    )(offsets, offsets, gt_nr_skel, gt_nr_skel, gt_dt_grad, gt_dt_grad, gt_labels)


NTILES = 32
PPT = BN // NTILES
BLOCK = 2048
NB = PPT // BLOCK
GR = 128
NG = BLOCK // GR
NL = GR // 16


def _sc_benefit_body(qref, offref, labref, outref, *s):
    offx = s[0:2]
    offy = s[2:4]
    lab = s[4:6]
    idx = (s[6:10], s[10:14])
    cbuf = (s[14:18], s[18:22])
    wx = s[22:24]
    wya = s[24:26]
    wyb = s[26:28]
    acc_v = s[28]
    sem_in = s[29]
    sem_g = s[30]

    cid = lax.axis_index("c")
    sid = lax.axis_index("s")
    wid = cid * 16 + sid
    base = wid * PPT
    b = lax.shift_right_logical(base, 18)
    bN = lax.shift_left(b, 18)
    f0_tile = base - bN
    off_base_tile = base + bN

    lanes = lax.iota(jnp.int32, 16)

    def inflow(ib, q):
        @pl.when(ib < NB)
        def _():
            blk = ib * BLOCK
            ox_start = pl.multiple_of(off_base_tile + blk, 8)
            oy_start = pl.multiple_of(off_base_tile + N + blk, 8)
            lb_start = pl.multiple_of(base + blk, 8)
            pltpu.async_copy(offref.at[pl.ds(ox_start, BLOCK)], offx[q], sem_in)
            pltpu.async_copy(offref.at[pl.ds(oy_start, BLOCK)], offy[q], sem_in)
            pltpu.async_copy(labref.at[pl.ds(lb_start, BLOCK)], lab[q], sem_in)

    def wait_in(q):
        pltpu.make_async_copy(offref.at[pl.ds(0, BLOCK)], offx[q], sem_in).wait()
        pltpu.make_async_copy(offref.at[pl.ds(0, BLOCK)], offy[q], sem_in).wait()
        pltpu.make_async_copy(labref.at[pl.ds(0, BLOCK)], lab[q], sem_in).wait()

    def pass1(ib, q):
        f_blk = f0_tile + ib * BLOCK
        offx_v, offy_v, lab_v = offx[q], offy[q], lab[q]
        idx0_v, idx1_v, idx2_v, idx3_v = idx[q]
        wx_v, wya_v, wyb_v = wx[q], wya[q], wyb[q]

        @plsc.parallel_loop(0, BLOCK // 16, unroll=4)
        def p1(g):
            l = g * 16
            ig = lax.shift_right_logical(g, 3)
            j = jnp.bitwise_and(g, NL - 1)
            ox = offx_v[pl.ds(l, 16)]
            oy = offy_v[pl.ds(l, 16)]
            lb = lab_v[pl.ds(l, 16)]
            f = f_blk + l + lanes
            xi = jnp.bitwise_and(f, W - 1)
            yi = lax.shift_right_logical(f, 9)
            px = xi.astype(jnp.float32) + ox
            py = yi.astype(jnp.float32) + oy
            px = jnp.minimum(jnp.maximum(px, 0.0), float(W - 1))
            py = jnp.minimum(jnp.maximum(py, 0.0), float(H - 1))
            x0 = px.astype(jnp.int32)
            y0 = py.astype(jnp.int32)
            wx1 = px - x0.astype(jnp.float32)
            wy1 = py - y0.astype(jnp.float32)
            fgm = jnp.where(lb > 0, 1.0, 0.0)
            gidx = bN + lax.shift_left(y0, 9) + x0
            sl16 = pl.ds(j * 16, 16)
            idx0_v[ig, sl16] = gidx
            idx1_v[ig, sl16] = gidx + 1
            idx2_v[ig, sl16] = gidx + W
            idx3_v[ig, sl16] = gidx + (W + 1)
            l_sl = pl.ds(l, 16)
            wx_v[l_sl] = wx1
            wya_v[l_sl] = (1.0 - wy1) * fgm
            wyb_v[l_sl] = wy1 * fgm

    def fire(q):
        idx0_v, idx1_v, idx2_v, idx3_v = idx[q]
        c0_v, c1_v, c2_v, c3_v = cbuf[q]

        def f1(ig, carry):
            dst = pl.ds(ig * GR, GR)
            pltpu.async_copy(qref.at[idx0_v.at[ig]], c0_v.at[dst], sem_g)
            pltpu.async_copy(qref.at[idx1_v.at[ig]], c1_v.at[dst], sem_g)
            pltpu.async_copy(qref.at[idx2_v.at[ig]], c2_v.at[dst], sem_g)
            pltpu.async_copy(qref.at[idx3_v.at[ig]], c3_v.at[dst], sem_g)
            return carry

        lax.fori_loop(0, NG, f1, 0)

    def drain_g(q):
        for cv in cbuf[q]:
            pltpu.make_async_copy(offref.at[pl.ds(0, BLOCK)], cv, sem_g).wait()

    def pass2(q, acc):
        c0_v, c1_v, c2_v, c3_v = cbuf[q]
        wx_v, wya_v, wyb_v = wx[q], wya[q], wyb[q]

        @plsc.parallel_loop(0, BLOCK // 16, unroll=4, carry=acc)
        def p2(j, acc):
            sl = pl.ds(j * 16, 16)
            wx1 = wx_v[sl]
            wx0 = 1.0 - wx1
            top = c0_v[sl] * wx0 + c1_v[sl] * wx1
            bot = c2_v[sl] * wx0 + c3_v[sl] * wx1
            return acc + top * wya_v[sl] + bot * wyb_v[sl]

        return p2

    acc = jnp.zeros((16,), jnp.float32)

    inflow(0, 0)
    wait_in(0)
    pass1(0, 0)
    fire(0)
    inflow(1, 1)

    def pair(k, acc):
        ibB = 2 * k + 1
        wait_in(1)
        pass1(ibB, 1)
        fire(1)
        inflow(ibB + 1, 0)
        drain_g(0)
        acc = pass2(0, acc)
        ibA = 2 * k + 2

        @pl.when(ibA < NB)
        def _():
            wait_in(0)
            pass1(ibA, 0)
            fire(0)

        inflow(ibA + 1, 1)
        drain_g(1)
        acc = pass2(1, acc)
        return acc

    acc = lax.fori_loop(0, NB // 2, pair, acc)

    acc_v[...] = acc
    pltpu.sync_copy(acc_v, outref.at[wid])


def _sc_benefit(img_pad, offs_flat, labels_flat):
    mesh = plsc.VectorSubcoreMesh(core_axis_name="c", subcore_axis_name="s")
    scratch = []
    for _ in range(2):
        scratch.append(pltpu.VMEM((BLOCK,), jnp.float32))
    for _ in range(2):
        scratch.append(pltpu.VMEM((BLOCK,), jnp.float32))
    for _ in range(2):
        scratch.append(pltpu.VMEM((BLOCK,), jnp.int32))
    for _ in range(8):
        scratch.append(pltpu.VMEM((NG, GR), jnp.int32))
    for _ in range(8):
        scratch.append(pltpu.VMEM((BLOCK,), jnp.float32))
    for _ in range(6):
        scratch.append(pltpu.VMEM((BLOCK,), jnp.float32))
    scratch.append(pltpu.VMEM((16,), jnp.float32))
    scratch.append(pltpu.SemaphoreType.DMA)
    scratch.append(pltpu.SemaphoreType.DMA)
    fn = functools.partial(
        pl.kernel, mesh=mesh,
        out_type=jax.ShapeDtypeStruct((NTILES, 16), jnp.float32),
        scratch_types=scratch,
    )(_sc_benefit_body)
    return fn(img_pad, offs_flat, labels_flat)


def kernel(offsets, gt_labels, gt_nr_skel, gt_dt_norm, gt_dt_grad):
    misc, sx, sy, cnt = _dense_terms(offsets, gt_labels, gt_nr_skel, gt_dt_grad)

    img_pad = jnp.concatenate(
        [gt_dt_norm.reshape(BN), jnp.zeros((1024,), jnp.float32)])
    partials = _sc_benefit(img_pad,
                           offsets.reshape(B * S * N),
                           gt_labels.reshape(BN).astype(jnp.int32))

    pull_sum = jnp.sum(misc[0])
    pen_sum = jnp.sum(misc[1])
    fg_sum = jnp.sum(misc[2])
    n_fg = jnp.maximum(fg_sum, 1.0)

    sums_x = jnp.sum(sx, axis=-1)
    sums_y = jnp.sum(sy, axis=-1)
    cnts = jnp.sum(cnt, axis=-1)

    mu = jnp.stack([sums_x, sums_y], axis=-1) / jnp.maximum(cnts, 1.0)[..., None]
    valid = ((cnts > 0) & (jnp.arange(K)[None, :] > 0)).astype(jnp.float32)
    dmu = jnp.sqrt(jnp.sum((mu[:, :, None, :] - mu[:, None, :, :]) ** 2, axis=-1) + 1e-8)
    pm = valid[:, :, None] * valid[:, None, :] * (1.0 - jnp.eye(K)[None])
    hinge = jnp.maximum(DELTA_PUSH - dmu, 0.0) ** 2
    l_push = jnp.sum(hinge * pm) / jnp.maximum(jnp.sum(pm), 1.0)

    ben_sum = fg_sum - jnp.sum(partials)

    l_pull = pull_sum / n_fg
    l_pen = pen_sum / n_fg
    l_ben = ben_sum / n_fg
    total = W_PULL * l_pull + W_PUSH * l_push + W_PEN * l_pen + W_BEN * l_ben
    return total, l_pull, l_push, l_pen, l_ben


# speedup vs baseline: 1.0030x; 1.0030x over previous
import functools

import jax
import jax.numpy as jnp
from jax import lax
from jax.experimental import pallas as pl
from jax.experimental.pallas import tpu as pltpu
from jax.experimental.pallas import tpu_sc as plsc

B, S, H, W = 4, 2, 512, 512
K = 8
N = H * W
BN = B * N
DELTA_PUSH = 20.0
W_PULL, W_PUSH, W_PEN, W_BEN = 1.0, 1.0, 1.0, 5.0

BH = 64
R = H // BH


def _dense_body(off_x, off_y, nr_x, nr_y, gr_x, gr_y, lab,
                misc_out, sx_out, sy_out, cnt_out,
                acc_misc, acc_sx, acc_sy, acc_cnt):
    b = pl.program_id(0)
    r = pl.program_id(1)

    @pl.when(jnp.logical_and(b == 0, r == 0))
    def _init():
        acc_misc[...] = jnp.zeros_like(acc_misc)

    @pl.when(r == 0)
    def _init_batch():
        acc_sx[...] = jnp.zeros_like(acc_sx)
        acc_sy[...] = jnp.zeros_like(acc_sy)
        acc_cnt[...] = jnp.zeros_like(acc_cnt)

    ox = off_x[0, 0]
    oy = off_y[0, 0]
    nx = nr_x[0, 0]
    ny = nr_y[0, 0]
    gx_ = gr_x[0, 0]
    gy_ = gr_y[0, 0]
    labs = lab[0]

    col = lax.broadcasted_iota(jnp.int32, (BH, W), 1).astype(jnp.float32)
    row = (lax.broadcasted_iota(jnp.int32, (BH, W), 0) + r * BH).astype(jnp.float32)
    ex = col + ox
    ey = row + oy

    fg = (labs > 0).astype(jnp.float32)

    d2 = (ex - nx) ** 2 + (ey - ny) ** 2 + 1e-8
    dist = jnp.sqrt(d2)
    acc_misc[0:1, :] += jnp.sum(dist * fg, axis=0, keepdims=True)

    off_n = jnp.sqrt(ox * ox + oy * oy + 1e-12)
    grad_n = jnp.sqrt(gx_ * gx_ + gy_ * gy_ + 1e-12)
    cos = (ox * gx_ + oy * gy_) / (off_n * grad_n + 1e-8)
    acc_misc[1:2, :] += jnp.sum((1.0 - cos) * fg, axis=0, keepdims=True)

    acc_misc[2:3, :] += jnp.sum(fg, axis=0, keepdims=True)

    for k in range(K):
        m = (labs == k).astype(jnp.float32)
        acc_sx[k:k + 1, :] += jnp.sum(m * ex, axis=0, keepdims=True)
        acc_sy[k:k + 1, :] += jnp.sum(m * ey, axis=0, keepdims=True)
        acc_cnt[k:k + 1, :] += jnp.sum(m, axis=0, keepdims=True)

    @pl.when(r == R - 1)
    def _flush_batch():
        sx_out[0] = acc_sx[...]
        sy_out[0] = acc_sy[...]
        cnt_out[0] = acc_cnt[...]

    @pl.when(jnp.logical_and(b == B - 1, r == R - 1))
    def _flush():
        misc_out[...] = acc_misc[...]


def _dense_terms(offsets, gt_labels, gt_nr_skel, gt_dt_grad):
    grid = (B, R)
    blk4 = (1, 1, BH, W)

    def chan(c):
        return pl.BlockSpec(blk4, lambda b, r, c=c: (b, c, r, 0))

    lab_spec = pl.BlockSpec((1, BH, W), lambda b, r: (b, r, 0))

    out_shapes = [
        jax.ShapeDtypeStruct((8, W), jnp.float32),
        jax.ShapeDtypeStruct((B, K, W), jnp.float32),
        jax.ShapeDtypeStruct((B, K, W), jnp.float32),
        jax.ShapeDtypeStruct((B, K, W), jnp.float32),
    ]
    out_specs = [
        pl.BlockSpec((8, W), lambda b, r: (0, 0)),
        pl.BlockSpec((1, K, W), lambda b, r: (b, 0, 0)),
        pl.BlockSpec((1, K, W), lambda b, r: (b, 0, 0)),
        pl.BlockSpec((1, K, W), lambda b, r: (b, 0, 0)),
    ]
    return pl.pallas_call(
        _dense_body,
        grid=grid,
        in_specs=[chan(0), chan(1), chan(0), chan(1), chan(0), chan(1), lab_spec],
        out_specs=out_specs,
        out_shape=out_shapes,
        scratch_shapes=[
            pltpu.VMEM((8, W), jnp.float32),
            pltpu.VMEM((K, W), jnp.float32),
            pltpu.VMEM((K, W), jnp.float32),
            pltpu.VMEM((K, W), jnp.float32),
        ],
    )(offsets, offsets, gt_nr_skel, gt_nr_skel, gt_dt_grad, gt_dt_grad, gt_labels)


NTILES = 32
PPT = BN // NTILES
BLOCK = 4096
NB = PPT // BLOCK
GR = 128
NG = BLOCK // GR
NL = GR // 16


def _sc_benefit_body(qref, offref, labref, outref, *s):
    offx = s[0:2]
    offy = s[2:4]
    lab = s[4:6]
    idx = (s[6:10], s[10:14])
    cbuf = (s[14:18], s[18:22])
    wx = s[22:24]
    wya = s[24:26]
    wyb = s[26:28]
    acc_v = s[28]
    sem_in = s[29]
    sem_g = s[30]

    cid = lax.axis_index("c")
    sid = lax.axis_index("s")
    wid = cid * 16 + sid
    base = wid * PPT
    b = lax.shift_right_logical(base, 18)
    bN = lax.shift_left(b, 18)
    f0_tile = base - bN
    off_base_tile = base + bN

    lanes = lax.iota(jnp.int32, 16)

    def inflow(ib, q):
        @pl.when(ib < NB)
        def _():
            blk = ib * BLOCK
            ox_start = pl.multiple_of(off_base_tile + blk, 8)
            oy_start = pl.multiple_of(off_base_tile + N + blk, 8)
            lb_start = pl.multiple_of(base + blk, 8)
            pltpu.async_copy(offref.at[pl.ds(ox_start, BLOCK)], offx[q], sem_in)
            pltpu.async_copy(offref.at[pl.ds(oy_start, BLOCK)], offy[q], sem_in)
            pltpu.async_copy(labref.at[pl.ds(lb_start, BLOCK)], lab[q], sem_in)

    def wait_in(q):
        pltpu.make_async_copy(offref.at[pl.ds(0, BLOCK)], offx[q], sem_in).wait()
        pltpu.make_async_copy(offref.at[pl.ds(0, BLOCK)], offy[q], sem_in).wait()
        pltpu.make_async_copy(labref.at[pl.ds(0, BLOCK)], lab[q], sem_in).wait()

    def pass1(ib, q):
        f_blk = f0_tile + ib * BLOCK
        offx_v, offy_v, lab_v = offx[q], offy[q], lab[q]
        idx0_v, idx1_v, idx2_v, idx3_v = idx[q]
        wx_v, wya_v, wyb_v = wx[q], wya[q], wyb[q]

        @plsc.parallel_loop(0, BLOCK // 16, unroll=4)
        def p1(g):
            l = g * 16
            ig = lax.shift_right_logical(g, 3)
            j = jnp.bitwise_and(g, NL - 1)
            ox = offx_v[pl.ds(l, 16)]
            oy = offy_v[pl.ds(l, 16)]
            lb = lab_v[pl.ds(l, 16)]
            f = f_blk + l + lanes
            xi = jnp.bitwise_and(f, W - 1)
            yi = lax.shift_right_logical(f, 9)
            px = xi.astype(jnp.float32) + ox
            py = yi.astype(jnp.float32) + oy
            px = jnp.minimum(jnp.maximum(px, 0.0), float(W - 1))
            py = jnp.minimum(jnp.maximum(py, 0.0), float(H - 1))
            x0 = px.astype(jnp.int32)
            y0 = py.astype(jnp.int32)
            wx1 = px - x0.astype(jnp.float32)
            wy1 = py - y0.astype(jnp.float32)
            fgm = jnp.where(lb > 0, 1.0, 0.0)
            gidx = bN + lax.shift_left(y0, 9) + x0
            sl16 = pl.ds(j * 16, 16)
            idx0_v[ig, sl16] = gidx
            idx1_v[ig, sl16] = gidx + 1
            idx2_v[ig, sl16] = gidx + W
            idx3_v[ig, sl16] = gidx + (W + 1)
            l_sl = pl.ds(l, 16)
            wx_v[l_sl] = wx1
            wya_v[l_sl] = (1.0 - wy1) * fgm
            wyb_v[l_sl] = wy1 * fgm

    def fire(q):
        idx0_v, idx1_v, idx2_v, idx3_v = idx[q]
        c0_v, c1_v, c2_v, c3_v = cbuf[q]

        def f1(ig, carry):
            dst = pl.ds(ig * GR, GR)
            pltpu.async_copy(qref.at[idx0_v.at[ig]], c0_v.at[dst], sem_g)
            pltpu.async_copy(qref.at[idx1_v.at[ig]], c1_v.at[dst], sem_g)
            pltpu.async_copy(qref.at[idx2_v.at[ig]], c2_v.at[dst], sem_g)
            pltpu.async_copy(qref.at[idx3_v.at[ig]], c3_v.at[dst], sem_g)
            return carry

        lax.fori_loop(0, NG, f1, 0)

    def drain_g(q):
        for cv in cbuf[q]:
            pltpu.make_async_copy(offref.at[pl.ds(0, BLOCK)], cv, sem_g).wait()

    def pass2(q, acc):
        c0_v, c1_v, c2_v, c3_v = cbuf[q]
        wx_v, wya_v, wyb_v = wx[q], wya[q], wyb[q]

        @plsc.parallel_loop(0, BLOCK // 16, unroll=4, carry=acc)
        def p2(j, acc):
            sl = pl.ds(j * 16, 16)
            wx1 = wx_v[sl]
            wx0 = 1.0 - wx1
            top = c0_v[sl] * wx0 + c1_v[sl] * wx1
            bot = c2_v[sl] * wx0 + c3_v[sl] * wx1
            return acc + top * wya_v[sl] + bot * wyb_v[sl]

        return p2

    acc = jnp.zeros((16,), jnp.float32)

    inflow(0, 0)
    wait_in(0)
    pass1(0, 0)
    fire(0)
    inflow(1, 1)

    def pair(k, acc):
        ibB = 2 * k + 1
        wait_in(1)
        pass1(ibB, 1)
        fire(1)
        inflow(ibB + 1, 0)
        drain_g(0)
        acc = pass2(0, acc)
        ibA = 2 * k + 2

        @pl.when(ibA < NB)
        def _():
            wait_in(0)
            pass1(ibA, 0)
            fire(0)

        inflow(ibA + 1, 1)
        drain_g(1)
        acc = pass2(1, acc)
        return acc

    acc = lax.fori_loop(0, NB // 2, pair, acc)

    acc_v[...] = acc
    pltpu.sync_copy(acc_v, outref.at[wid])


def _sc_benefit(img_pad, offs_flat, labels_flat):
    mesh = plsc.VectorSubcoreMesh(core_axis_name="c", subcore_axis_name="s")
    scratch = []
    for _ in range(2):
        scratch.append(pltpu.VMEM((BLOCK,), jnp.float32))
    for _ in range(2):
        scratch.append(pltpu.VMEM((BLOCK,), jnp.float32))
    for _ in range(2):
        scratch.append(pltpu.VMEM((BLOCK,), jnp.int32))
    for _ in range(8):
        scratch.append(pltpu.VMEM((NG, GR), jnp.int32))
    for _ in range(8):
        scratch.append(pltpu.VMEM((BLOCK,), jnp.float32))
    for _ in range(6):
        scratch.append(pltpu.VMEM((BLOCK,), jnp.float32))
    scratch.append(pltpu.VMEM((16,), jnp.float32))
    scratch.append(pltpu.SemaphoreType.DMA)
    scratch.append(pltpu.SemaphoreType.DMA)
    fn = functools.partial(
        pl.kernel, mesh=mesh,
        out_type=jax.ShapeDtypeStruct((NTILES, 16), jnp.float32),
        scratch_types=scratch,
    )(_sc_benefit_body)
    return fn(img_pad, offs_flat, labels_flat)


def kernel(offsets, gt_labels, gt_nr_skel, gt_dt_norm, gt_dt_grad):
    misc, sx, sy, cnt = _dense_terms(offsets, gt_labels, gt_nr_skel, gt_dt_grad)

    img_pad = jnp.concatenate(
        [gt_dt_norm.reshape(BN), jnp.zeros((1024,), jnp.float32)])
    partials = _sc_benefit(img_pad,
                           offsets.reshape(B * S * N),
                           gt_labels.reshape(BN).astype(jnp.int32))

    pull_sum = jnp.sum(misc[0])
    pen_sum = jnp.sum(misc[1])
    fg_sum = jnp.sum(misc[2])
    n_fg = jnp.maximum(fg_sum, 1.0)

    sums_x = jnp.sum(sx, axis=-1)
    sums_y = jnp.sum(sy, axis=-1)
    cnts = jnp.sum(cnt, axis=-1)

    mu = jnp.stack([sums_x, sums_y], axis=-1) / jnp.maximum(cnts, 1.0)[..., None]
    valid = ((cnts > 0) & (jnp.arange(K)[None, :] > 0)).astype(jnp.float32)
    dmu = jnp.sqrt(jnp.sum((mu[:, :, None, :] - mu[:, None, :, :]) ** 2, axis=-1) + 1e-8)
    pm = valid[:, :, None] * valid[:, None, :] * (1.0 - jnp.eye(K)[None])
    hinge = jnp.maximum(DELTA_PUSH - dmu, 0.0) ** 2
    l_push = jnp.sum(hinge * pm) / jnp.maximum(jnp.sum(pm), 1.0)

    ben_sum = fg_sum - jnp.sum(partials)

    l_pull = pull_sum / n_fg
    l_pen = pen_sum / n_fg
    l_ben = ben_sum / n_fg
    total = W_PULL * l_pull + W_PUSH * l_push + W_PEN * l_pen + W_BEN * l_ben
    return total, l_pull, l_push, l_pen, l_ben


# speedup vs baseline: 2.8881x; 2.8796x over previous
import functools

import jax
import jax.numpy as jnp
from jax import lax
from jax.experimental import pallas as pl
from jax.experimental.pallas import tpu as pltpu
from jax.experimental.pallas import tpu_sc as plsc

B, S, H, W = 4, 2, 512, 512
K = 8
N = H * W
BN = B * N
DELTA_PUSH = 20.0
W_PULL, W_PUSH, W_PEN, W_BEN = 1.0, 1.0, 1.0, 5.0

BH = 64
R = H // BH


def _dense_body(off_x, off_y, nr_x, nr_y, gr_x, gr_y, lab,
                misc_out, sx_out, sy_out, cnt_out,
                acc_misc, acc_sx, acc_sy, acc_cnt):
    b = pl.program_id(0)
    r = pl.program_id(1)

    @pl.when(jnp.logical_and(b == 0, r == 0))
    def _init():
        acc_misc[...] = jnp.zeros_like(acc_misc)

    @pl.when(r == 0)
    def _init_batch():
        acc_sx[...] = jnp.zeros_like(acc_sx)
        acc_sy[...] = jnp.zeros_like(acc_sy)
        acc_cnt[...] = jnp.zeros_like(acc_cnt)

    ox = off_x[0, 0]
    oy = off_y[0, 0]
    nx = nr_x[0, 0]
    ny = nr_y[0, 0]
    gx_ = gr_x[0, 0]
    gy_ = gr_y[0, 0]
    labs = lab[0]

    col = lax.broadcasted_iota(jnp.int32, (BH, W), 1).astype(jnp.float32)
    row = (lax.broadcasted_iota(jnp.int32, (BH, W), 0) + r * BH).astype(jnp.float32)
    ex = col + ox
    ey = row + oy

    fg = (labs > 0).astype(jnp.float32)

    d2 = (ex - nx) ** 2 + (ey - ny) ** 2 + 1e-8
    dist = jnp.sqrt(d2)
    acc_misc[0:1, :] += jnp.sum(dist * fg, axis=0, keepdims=True)

    off_n = jnp.sqrt(ox * ox + oy * oy + 1e-12)
    grad_n = jnp.sqrt(gx_ * gx_ + gy_ * gy_ + 1e-12)
    cos = (ox * gx_ + oy * gy_) / (off_n * grad_n + 1e-8)
    acc_misc[1:2, :] += jnp.sum((1.0 - cos) * fg, axis=0, keepdims=True)

    acc_misc[2:3, :] += jnp.sum(fg, axis=0, keepdims=True)

    for k in range(K):
        m = (labs == k).astype(jnp.float32)
        acc_sx[k:k + 1, :] += jnp.sum(m * ex, axis=0, keepdims=True)
        acc_sy[k:k + 1, :] += jnp.sum(m * ey, axis=0, keepdims=True)
        acc_cnt[k:k + 1, :] += jnp.sum(m, axis=0, keepdims=True)

    @pl.when(r == R - 1)
    def _flush_batch():
        sx_out[0] = acc_sx[...]
        sy_out[0] = acc_sy[...]
        cnt_out[0] = acc_cnt[...]

    @pl.when(jnp.logical_and(b == B - 1, r == R - 1))
    def _flush():
        misc_out[...] = acc_misc[...]


def _dense_terms(offsets, gt_labels, gt_nr_skel, gt_dt_grad):
    grid = (B, R)
    blk4 = (1, 1, BH, W)

    def chan(c):
        return pl.BlockSpec(blk4, lambda b, r, c=c: (b, c, r, 0))

    lab_spec = pl.BlockSpec((1, BH, W), lambda b, r: (b, r, 0))

    out_shapes = [
        jax.ShapeDtypeStruct((8, W), jnp.float32),
        jax.ShapeDtypeStruct((B, K, W), jnp.float32),
        jax.ShapeDtypeStruct((B, K, W), jnp.float32),
        jax.ShapeDtypeStruct((B, K, W), jnp.float32),
    ]
    out_specs = [
        pl.BlockSpec((8, W), lambda b, r: (0, 0)),
        pl.BlockSpec((1, K, W), lambda b, r: (b, 0, 0)),
        pl.BlockSpec((1, K, W), lambda b, r: (b, 0, 0)),
        pl.BlockSpec((1, K, W), lambda b, r: (b, 0, 0)),
    ]
    return pl.pallas_call(
        _dense_body,
        grid=grid,
        in_specs=[chan(0), chan(1), chan(0), chan(1), chan(0), chan(1), lab_spec],
        out_specs=out_specs,
        out_shape=out_shapes,
        scratch_shapes=[
            pltpu.VMEM((8, W), jnp.float32),
            pltpu.VMEM((K, W), jnp.float32),
            pltpu.VMEM((K, W), jnp.float32),
            pltpu.VMEM((K, W), jnp.float32),
        ],
    )(offsets, offsets, gt_nr_skel, gt_nr_skel, gt_dt_grad, gt_dt_grad, gt_labels)


NTILES = 32
PPT = BN // NTILES
BLOCK = 2048
NB = PPT // BLOCK
GR = 128
NG = BLOCK // GR
NL = GR // 16


def _sc_benefit_body(qref, offref, labref, outref, *s):
    offx = s[0:2]
    offy = s[2:4]
    lab = s[4:6]
    idx = (s[6:10], s[10:14])
    cbuf = (s[14:18], s[18:22])
    wx = s[22:24]
    wya = s[24:26]
    wyb = s[26:28]
    acc_v = s[28]
    simg = s[29]
    sem_in = s[30]
    sem_g = s[31]

    cid = lax.axis_index("c")
    sid = lax.axis_index("s")
    wid = cid * 16 + sid
    base = wid * PPT
    b = lax.shift_right_logical(base, 18)
    bN = lax.shift_left(b, 18)
    f0_tile = base - bN
    off_base_tile = base + bN
    slab = 2 * N + 1024
    simg_base = cid * slab + lax.shift_left(jnp.bitwise_and(b, 1), 18)

    lanes = lax.iota(jnp.int32, 16)

    def inflow(ib, q):
        @pl.when(ib < NB)
        def _():
            blk = ib * BLOCK
            ox_start = pl.multiple_of(off_base_tile + blk, 8)
            oy_start = pl.multiple_of(off_base_tile + N + blk, 8)
            lb_start = pl.multiple_of(base + blk, 8)
            pltpu.async_copy(offref.at[pl.ds(ox_start, BLOCK)], offx[q], sem_in)
            pltpu.async_copy(offref.at[pl.ds(oy_start, BLOCK)], offy[q], sem_in)
            pltpu.async_copy(labref.at[pl.ds(lb_start, BLOCK)], lab[q], sem_in)

    def wait_in(q):
        pltpu.make_async_copy(offref.at[pl.ds(0, BLOCK)], offx[q], sem_in).wait()
        pltpu.make_async_copy(offref.at[pl.ds(0, BLOCK)], offy[q], sem_in).wait()
        pltpu.make_async_copy(labref.at[pl.ds(0, BLOCK)], lab[q], sem_in).wait()

    def pass1(ib, q):
        f_blk = f0_tile + ib * BLOCK
        offx_v, offy_v, lab_v = offx[q], offy[q], lab[q]
        idx0_v, idx1_v, idx2_v, idx3_v = idx[q]
        wx_v, wya_v, wyb_v = wx[q], wya[q], wyb[q]

        @plsc.parallel_loop(0, BLOCK // 16, unroll=4)
        def p1(g):
            l = g * 16
            ig = lax.shift_right_logical(g, 3)
            j = jnp.bitwise_and(g, NL - 1)
            ox = offx_v[pl.ds(l, 16)]
            oy = offy_v[pl.ds(l, 16)]
            lb = lab_v[pl.ds(l, 16)]
            f = f_blk + l + lanes
            xi = jnp.bitwise_and(f, W - 1)
            yi = lax.shift_right_logical(f, 9)
            px = xi.astype(jnp.float32) + ox
            py = yi.astype(jnp.float32) + oy
            px = jnp.minimum(jnp.maximum(px, 0.0), float(W - 1))
            py = jnp.minimum(jnp.maximum(py, 0.0), float(H - 1))
            x0 = px.astype(jnp.int32)
            y0 = py.astype(jnp.int32)
            wx1 = px - x0.astype(jnp.float32)
            wy1 = py - y0.astype(jnp.float32)
            fgm = jnp.where(lb > 0, 1.0, 0.0)
            gidx = simg_base + lax.shift_left(y0, 9) + x0
            sl16 = pl.ds(j * 16, 16)
            idx0_v[ig, sl16] = gidx
            idx1_v[ig, sl16] = gidx + 1
            idx2_v[ig, sl16] = gidx + W
            idx3_v[ig, sl16] = gidx + (W + 1)
            l_sl = pl.ds(l, 16)
            wx_v[l_sl] = wx1
            wya_v[l_sl] = (1.0 - wy1) * fgm
            wyb_v[l_sl] = wy1 * fgm

    def fire(q):
        idx0_v, idx1_v, idx2_v, idx3_v = idx[q]
        c0_v, c1_v, c2_v, c3_v = cbuf[q]

        def f1(ig, carry):
            dst = pl.ds(ig * GR, GR)
            pltpu.async_copy(simg.at[idx0_v.at[ig]], c0_v.at[dst], sem_g)
            pltpu.async_copy(simg.at[idx1_v.at[ig]], c1_v.at[dst], sem_g)
            pltpu.async_copy(simg.at[idx2_v.at[ig]], c2_v.at[dst], sem_g)
            pltpu.async_copy(simg.at[idx3_v.at[ig]], c3_v.at[dst], sem_g)
            return carry

        lax.fori_loop(0, NG, f1, 0)

    def drain_g(q):
        for cv in cbuf[q]:
            pltpu.make_async_copy(offref.at[pl.ds(0, BLOCK)], cv, sem_g).wait()

    def pass2(q, acc):
        c0_v, c1_v, c2_v, c3_v = cbuf[q]
        wx_v, wya_v, wyb_v = wx[q], wya[q], wyb[q]

        @plsc.parallel_loop(0, BLOCK // 16, unroll=4, carry=acc)
        def p2(j, acc):
            sl = pl.ds(j * 16, 16)
            wx1 = wx_v[sl]
            wx0 = 1.0 - wx1
            top = c0_v[sl] * wx0 + c1_v[sl] * wx1
            bot = c2_v[sl] * wx0 + c3_v[sl] * wx1
            return acc + top * wya_v[sl] + bot * wyb_v[sl]

        return p2

    acc = jnp.zeros((16,), jnp.float32)

    inflow(0, 0)

    @pl.when(sid == 0)
    def _stage():
        src_start = pl.multiple_of(cid * (2 * N), 8)
        dst_start = pl.multiple_of(cid * slab, 8)
        pltpu.sync_copy(qref.at[pl.ds(src_start, slab)],
                        simg.at[pl.ds(dst_start, slab)])

    plsc.subcore_barrier()

    wait_in(0)
    pass1(0, 0)
    fire(0)
    inflow(1, 1)

    def pair(k, acc):
        ibB = 2 * k + 1
        wait_in(1)
        pass1(ibB, 1)
        fire(1)
        inflow(ibB + 1, 0)
        drain_g(0)
        acc = pass2(0, acc)
        ibA = 2 * k + 2

        @pl.when(ibA < NB)
        def _():
            wait_in(0)
            pass1(ibA, 0)
            fire(0)

        inflow(ibA + 1, 1)
        drain_g(1)
        acc = pass2(1, acc)
        return acc

    acc = lax.fori_loop(0, NB // 2, pair, acc)

    acc_v[...] = acc
    pltpu.sync_copy(acc_v, outref.at[wid])


def _sc_benefit(img_pad, offs_flat, labels_flat):
    mesh = plsc.VectorSubcoreMesh(core_axis_name="c", subcore_axis_name="s")
    scratch = []
    for _ in range(2):
        scratch.append(pltpu.VMEM((BLOCK,), jnp.float32))
    for _ in range(2):
        scratch.append(pltpu.VMEM((BLOCK,), jnp.float32))
    for _ in range(2):
        scratch.append(pltpu.VMEM((BLOCK,), jnp.int32))
    for _ in range(8):
        scratch.append(pltpu.VMEM((NG, GR), jnp.int32))
    for _ in range(8):
        scratch.append(pltpu.VMEM((BLOCK,), jnp.float32))
    for _ in range(6):
        scratch.append(pltpu.VMEM((BLOCK,), jnp.float32))
    scratch.append(pltpu.VMEM((16,), jnp.float32))
    scratch.append(pltpu.VMEM_SHARED((BN + 1024,), jnp.float32))
    scratch.append(pltpu.SemaphoreType.DMA)
    scratch.append(pltpu.SemaphoreType.DMA)
    fn = functools.partial(
        pl.kernel, mesh=mesh,
        out_type=jax.ShapeDtypeStruct((NTILES, 16), jnp.float32),
        scratch_types=scratch,
    )(_sc_benefit_body)
    return fn(img_pad, offs_flat, labels_flat)


def kernel(offsets, gt_labels, gt_nr_skel, gt_dt_norm, gt_dt_grad):
    misc, sx, sy, cnt = _dense_terms(offsets, gt_labels, gt_nr_skel, gt_dt_grad)

    img_pad = jnp.concatenate(
        [gt_dt_norm.reshape(BN), jnp.zeros((1024,), jnp.float32)])
    partials = _sc_benefit(img_pad,
                           offsets.reshape(B * S * N),
                           gt_labels.reshape(BN).astype(jnp.int32))

    pull_sum = jnp.sum(misc[0])
    pen_sum = jnp.sum(misc[1])
    fg_sum = jnp.sum(misc[2])
    n_fg = jnp.maximum(fg_sum, 1.0)

    sums_x = jnp.sum(sx, axis=-1)
    sums_y = jnp.sum(sy, axis=-1)
    cnts = jnp.sum(cnt, axis=-1)

    mu = jnp.stack([sums_x, sums_y], axis=-1) / jnp.maximum(cnts, 1.0)[..., None]
    valid = ((cnts > 0) & (jnp.arange(K)[None, :] > 0)).astype(jnp.float32)
    dmu = jnp.sqrt(jnp.sum((mu[:, :, None, :] - mu[:, None, :, :]) ** 2, axis=-1) + 1e-8)
    pm = valid[:, :, None] * valid[:, None, :] * (1.0 - jnp.eye(K)[None])
    hinge = jnp.maximum(DELTA_PUSH - dmu, 0.0) ** 2
    l_push = jnp.sum(hinge * pm) / jnp.maximum(jnp.sum(pm), 1.0)

    ben_sum = fg_sum - jnp.sum(partials)

    l_pull = pull_sum / n_fg
    l_pen = pen_sum / n_fg
    l_ben = ben_sum / n_fg
    total = W_PULL * l_pull + W_PUSH * l_push + W_PEN * l_pen + W_BEN * l_ben
    return total, l_pull, l_push, l_pen, l_ben
